# baseline (speedup 1.0000x reference)
import jax
import jax.numpy as jnp
from jax import lax
from jax.experimental import pallas as pl
from jax.experimental.pallas import tpu as pltpu
from jax.experimental.pallas import tpu_sc as plsc

ROWS = 64
COLS = 8192
K = 3
LANES = 16
NUM_CORES = 2
NUM_SUBCORES = 16
NUM_WORKERS = NUM_CORES * NUM_SUBCORES
ROWS_PER_WORKER = ROWS // NUM_WORKERS
ROWS_PER_CORE = ROWS // NUM_CORES
CHUNKS = COLS // LANES
STREAMS = 4
STEPS = CHUNKS // STREAMS
PACK = ROWS_PER_CORE * K
PACK_GROUPS = PACK // LANES


def _insert(acc, cvals, cidx):
    v1, i1, v2, i2, v3, i3 = acc
    gt1 = cvals > v1
    t = jnp.minimum(cvals, v1)
    it = jnp.where(gt1, i1, cidx)
    v1 = jnp.maximum(cvals, v1)
    i1 = jnp.where(gt1, cidx, i1)
    gt2 = t > v2
    t2 = jnp.minimum(t, v2)
    it2 = jnp.where(gt2, i2, it)
    v2 = jnp.maximum(t, v2)
    i2 = jnp.where(gt2, it, i2)
    gt3 = t2 > v3
    v3 = jnp.maximum(t2, v3)
    i3 = jnp.where(gt3, it2, i3)
    return v1, i1, v2, i2, v3, i3


def _merge(a, b):
    for lv in range(3):
        a = _insert(a, b[2 * lv], b[2 * lv + 1])
    return a


def _body(x_hbm, vals_hbm, idx_hbm,
          rows_v, wi_v, resv_v, resi_v, packv_v, packi_v,
          stagev_sh, stagei_sh, sem):
    c = lax.axis_index("c")
    s = lax.axis_index("s")
    wid = c * NUM_SUBCORES + s

    lane = lax.broadcasted_iota(jnp.int32, (LANES,), 0)
    neg = jnp.full((LANES,), -jnp.inf, jnp.float32)
    zero_i = jnp.zeros((LANES,), jnp.int32)
    big = jnp.full((LANES,), jnp.int32(2**30), jnp.int32)

    base = wid * ROWS_PER_WORKER
    cps = [
        pltpu.make_async_copy(
            x_hbm.at[base + r], rows_v.at[pl.ds(r * COLS, COLS)], sem)
        for r in range(ROWS_PER_WORKER)
    ]
    for cp in cps:
        cp.start()
    for cp in cps:
        cp.wait()

    def row_body(r, _):
        roff = r * COLS
        init = tuple((neg, zero_i, neg, zero_i, neg, zero_i)[i % 6]
                     for i in range(6 * STREAMS))

        def step(j, carry):
            accs = [carry[6 * q:6 * q + 6] for q in range(STREAMS)]
            out = []
            coff = j * (STREAMS * LANES)
            for q in range(STREAMS):
                cvals = rows_v[pl.ds(roff + coff + q * LANES, LANES)]
                cidx = lane + (coff + q * LANES)
                out.extend(_insert(accs[q], cvals, cidx))
            return tuple(out)

        flat = lax.fori_loop(0, STEPS, step, init)
        accs = [flat[6 * q:6 * q + 6] for q in range(STREAMS)]
        while len(accs) > 1:
            accs = [_merge(accs[i], accs[i + 1])
                    for i in range(0, len(accs), 2)]
        v1, i1, v2, i2, v3, i3 = accs[0]

        out_v, out_i = [], []
        for _k in range(K):
            m = jnp.max(v1)
            sel = jnp.min(jnp.where(v1 == m, i1, big))
            out_v.append(m)
            out_i.append(sel)
            hit = (v1 == m) & (i1 == sel)
            v1 = jnp.where(hit, v2, v1)
            i1 = jnp.where(hit, i2, i1)
            v2 = jnp.where(hit, v3, v2)
            i2 = jnp.where(hit, i3, i2)
            v3 = jnp.where(hit, neg, v3)

        resv = jnp.where(lane == 0, out_v[0],
                         jnp.where(lane == 1, out_v[1],
                                   jnp.where(lane == 2, out_v[2], 0.0)))
        resi = jnp.where(lane == 0, out_i[0],
                         jnp.where(lane == 1, out_i[1],
                                   jnp.where(lane == 2, out_i[2], 0)))
        resv_v[...] = resv.astype(jnp.float32)
        resi_v[...] = resi.astype(jnp.int32)
        soff = (s * ROWS_PER_WORKER + r) * LANES
        pltpu.sync_copy(resv_v, stagev_sh.at[pl.ds(soff, LANES)])
        pltpu.sync_copy(resi_v, stagei_sh.at[pl.ds(soff, LANES)])
        return 0

    lax.fori_loop(0, ROWS_PER_WORKER, row_body, 0)
    plsc.subcore_barrier()

    @pl.when(s == 0)
    def _compact():
        stage_words = ROWS_PER_CORE * LANES
        wv = rows_v.at[pl.ds(0, stage_words)]
        pltpu.sync_copy(stagev_sh, wv)
        pltpu.sync_copy(stagei_sh, wi_v)
        for g in range(PACK_GROUPS):
            p = lane + g * LANES
            row = p // K
            slot = p - row * K
            src = row * LANES + slot
            gv = plsc.load_gather(wv, [src])
            gi = plsc.load_gather(wi_v, [src])
            packv_v[pl.ds(g * LANES, LANES)] = gv
            packi_v[pl.ds(g * LANES, LANES)] = gi
        pltpu.sync_copy(packv_v, vals_hbm.at[pl.ds(c * PACK, PACK)])
        pltpu.sync_copy(packi_v, idx_hbm.at[pl.ds(c * PACK, PACK)])


@jax.jit
def _topk_sc(x):
    mesh = plsc.VectorSubcoreMesh(core_axis_name="c", subcore_axis_name="s")
    fn = pl.kernel(
        _body,
        out_type=(
            jax.ShapeDtypeStruct((ROWS * K,), jnp.float32),
            jax.ShapeDtypeStruct((ROWS * K,), jnp.int32),
        ),
        mesh=mesh,
        scratch_types=[
            pltpu.VMEM((ROWS_PER_WORKER * COLS,), jnp.float32),
            pltpu.VMEM((ROWS_PER_CORE * LANES,), jnp.int32),
            pltpu.VMEM((LANES,), jnp.float32),
            pltpu.VMEM((LANES,), jnp.int32),
            pltpu.VMEM((ROWS_PER_CORE * K,), jnp.float32),
            pltpu.VMEM((ROWS_PER_CORE * K,), jnp.int32),
            pltpu.VMEM_SHARED((ROWS_PER_CORE * LANES,), jnp.float32),
            pltpu.VMEM_SHARED((ROWS_PER_CORE * LANES,), jnp.int32),
            pltpu.SemaphoreType.DMA,
        ],
        compiler_params=pltpu.CompilerParams(needs_layout_passes=False),
    )
    return fn(x)


def kernel(x):
    vals_p, idx_p = _topk_sc(x)
    return vals_p.reshape(ROWS, K), idx_p.reshape(ROWS, K)


# speedup vs baseline: 1.0432x; 1.0432x over previous
import jax
import jax.numpy as jnp
from jax import lax
from jax.experimental import pallas as pl
from jax.experimental.pallas import tpu as pltpu
from jax.experimental.pallas import tpu_sc as plsc

ROWS = 64
COLS = 8192
K = 3
LANES = 16
NUM_CORES = 2
NUM_SUBCORES = 16
NUM_WORKERS = NUM_CORES * NUM_SUBCORES
ROWS_PER_WORKER = ROWS // NUM_WORKERS
CHUNKS = COLS // LANES
STREAMS = 4
STEPS = CHUNKS // STREAMS


def _insert(acc, cvals, cidx):
    v1, i1, v2, i2, v3, i3 = acc
    gt1 = cvals > v1
    t = jnp.minimum(cvals, v1)
    it = jnp.where(gt1, i1, cidx)
    v1 = jnp.maximum(cvals, v1)
    i1 = jnp.where(gt1, cidx, i1)
    gt2 = t > v2
    t2 = jnp.minimum(t, v2)
    it2 = jnp.where(gt2, i2, it)
    v2 = jnp.maximum(t, v2)
    i2 = jnp.where(gt2, it, i2)
    gt3 = t2 > v3
    v3 = jnp.maximum(t2, v3)
    i3 = jnp.where(gt3, it2, i3)
    return v1, i1, v2, i2, v3, i3


def _merge(a, b):
    for lv in range(3):
        a = _insert(a, b[2 * lv], b[2 * lv + 1])
    return a


def _body(x_hbm, out_hbm, rows_v, resv_v, resi_v, sem):
    c = lax.axis_index("c")
    s = lax.axis_index("s")
    wid = s * NUM_CORES + c

    lane = lax.broadcasted_iota(jnp.int32, (LANES,), 0)
    neg = jnp.full((LANES,), -jnp.inf, jnp.float32)
    zero_i = jnp.zeros((LANES,), jnp.int32)
    big = jnp.full((LANES,), jnp.int32(2**30), jnp.int32)

    base = wid * ROWS_PER_WORKER
    cps = [
        pltpu.make_async_copy(
            x_hbm.at[base + r], rows_v.at[pl.ds(r * COLS, COLS)], sem)
        for r in range(ROWS_PER_WORKER)
    ]
    for cp in cps:
        cp.start()
    for cp in cps:
        cp.wait()

    def row_body(r, _):
        roff = r * COLS
        init = tuple((neg, zero_i, neg, zero_i, neg, zero_i)[i % 6]
                     for i in range(6 * STREAMS))

        def step(j, carry):
            accs = [carry[6 * q:6 * q + 6] for q in range(STREAMS)]
            out = []
            coff = j * (STREAMS * LANES)
            for q in range(STREAMS):
                cvals = rows_v[pl.ds(roff + coff + q * LANES, LANES)]
                cidx = lane + (coff + q * LANES)
                out.extend(_insert(accs[q], cvals, cidx))
            return tuple(out)

        flat = lax.fori_loop(0, STEPS, step, init)
        accs = [flat[6 * q:6 * q + 6] for q in range(STREAMS)]
        while len(accs) > 1:
            accs = [_merge(accs[i], accs[i + 1])
                    for i in range(0, len(accs), 2)]
        v1, i1, v2, i2, v3, i3 = accs[0]

        out_v, out_i = [], []
        for _k in range(K):
            m = jnp.max(v1)
            sel = jnp.min(jnp.where(v1 == m, i1, big))
            out_v.append(m)
            out_i.append(sel)
            hit = (v1 == m) & (i1 == sel)
            v1 = jnp.where(hit, v2, v1)
            i1 = jnp.where(hit, i2, i1)
            v2 = jnp.where(hit, v3, v2)
            i2 = jnp.where(hit, i3, i2)
            v3 = jnp.where(hit, neg, v3)

        resv = jnp.where(lane == 0, out_v[0],
                         jnp.where(lane == 1, out_v[1],
                                   jnp.where(lane == 2, out_v[2], 0.0)))
        resi = jnp.where(lane == 0, out_i[0],
                         jnp.where(lane == 1, out_i[1],
                                   jnp.where(lane == 2, out_i[2], 0)))
        resv_v[...] = plsc.bitcast(resv.astype(jnp.float32), jnp.int32)
        resi_v[...] = resi.astype(jnp.int32)
        pltpu.sync_copy(resv_v, out_hbm.at[base + r])
        pltpu.sync_copy(resi_v, out_hbm.at[ROWS + base + r])
        return 0

    lax.fori_loop(0, ROWS_PER_WORKER, row_body, 0)


@jax.jit
def _topk_sc(x):
    mesh = plsc.VectorSubcoreMesh(core_axis_name="c", subcore_axis_name="s")
    fn = pl.kernel(
        _body,
        out_type=jax.ShapeDtypeStruct((2 * ROWS, LANES), jnp.int32),
        mesh=mesh,
        scratch_types=[
            pltpu.VMEM((ROWS_PER_WORKER * COLS,), jnp.float32),
            pltpu.VMEM((LANES,), jnp.int32),
            pltpu.VMEM((LANES,), jnp.int32),
            pltpu.SemaphoreType.DMA,
        ],
        compiler_params=pltpu.CompilerParams(needs_layout_passes=False),
    )
    return fn(x)


def kernel(x):
    out = _topk_sc(x)
    vals = lax.bitcast_convert_type(out[:ROWS, :K], jnp.float32)
    idx = out[ROWS:, :K]
    return vals, idx
